# baseline (speedup 1.0000x reference)
import functools

import jax
import jax.numpy as jnp
from jax import lax
from jax.experimental import pallas as pl
from jax.experimental.pallas import tpu as pltpu
from jax.experimental.pallas import tpu_sc as plsc

N = 10000
E = 320000
D = 128
H = 128
NUM_LEVELS = 3
NUM_PROC = 9

NC = 2
NS = 16
NW = NC * NS
CH = 128
CPW = 80
EPAD = NW * CPW * CH
NP = 10240
ROWS_PER_TILE = NP // NS


def _sc_body(x_hbm, src_hbm, dst_hbm, zz_hbm, out_hbm,
             sidx0, sidx1, didx0, didx1, rows0, rows1, aggr_sh,
             semr0, semr1, semd0, semd1, sems0, sems1):
    c = lax.axis_index("c")
    s = lax.axis_index("s")
    wid = s * NC + c
    pltpu.sync_copy(zz_hbm.at[pl.ds(s * ROWS_PER_TILE, ROWS_PER_TILE)],
                    aggr_sh.at[pl.ds(s * ROWS_PER_TILE, ROWS_PER_TILE)])
    plsc.subcore_barrier()

    ebase = wid * CPW * CH
    rows = (rows0, rows1)
    sidx = (sidx0, sidx1)
    didx = (didx0, didx1)
    semr = (semr0, semr1)
    semd = (semd0, semd1)
    sems = (sems0, sems1)

    def idx_fetch(kk, b):
        pltpu.async_copy(src_hbm.at[pl.ds(ebase + kk * CH, CH)],
                         sidx[b], sems[b])
        pltpu.async_copy(dst_hbm.at[pl.ds(ebase + kk * CH, CH)],
                         didx[b], semd[b])

    idx_fetch(0, 0)
    idx_fetch(1, 1)
    pltpu.make_async_copy(src_hbm.at[pl.ds(ebase, CH)], sidx0, sems0).wait()
    pltpu.async_copy(x_hbm.at[sidx0], rows0, semr0)

    def step(i, carry):
        for b in range(2):
            kk = 2 * i + b
            nb = b ^ 1

            @pl.when(kk + 1 < CPW)
            def _():
                pltpu.make_async_copy(
                    src_hbm.at[pl.ds(ebase + (kk + 1) * CH, CH)],
                    sidx[nb], sems[nb]).wait()
                pltpu.async_copy(x_hbm.at[sidx[nb]], rows[nb], semr[nb])

            pltpu.make_async_copy(dst_hbm.at[pl.ds(ebase + kk * CH, CH)],
                                  didx[b], semd[b]).wait()
            pltpu.make_async_copy(x_hbm.at[sidx[b]], rows[b], semr[b]).wait()
            pltpu.sync_copy(rows[b], aggr_sh.at[didx[b]], add=True)

            @pl.when(kk + 2 < CPW)
            def _():
                idx_fetch(kk + 2, b)
        return carry

    lax.fori_loop(0, CPW // 2, step, 0)

    plsc.subcore_barrier()
    pltpu.sync_copy(aggr_sh.at[pl.ds(s * ROWS_PER_TILE, ROWS_PER_TILE)],
                    out_hbm.at[c, pl.ds(s * ROWS_PER_TILE, ROWS_PER_TILE)])


@functools.cache
def _sc_segment_sum():
    return pl.kernel(
        _sc_body,
        out_type=jax.ShapeDtypeStruct((NC, NP, D), jnp.float32),
        mesh=plsc.VectorSubcoreMesh(core_axis_name="c", subcore_axis_name="s",
                                    num_cores=NC, num_subcores=NS),
        scratch_types=[
            pltpu.VMEM((CH,), jnp.int32),
            pltpu.VMEM((CH,), jnp.int32),
            pltpu.VMEM((CH,), jnp.int32),
            pltpu.VMEM((CH,), jnp.int32),
            pltpu.VMEM((CH, D), jnp.float32),
            pltpu.VMEM((CH, D), jnp.float32),
            pltpu.VMEM_SHARED((NP, D), jnp.float32),
            pltpu.SemaphoreType.DMA,
            pltpu.SemaphoreType.DMA,
            pltpu.SemaphoreType.DMA,
            pltpu.SemaphoreType.DMA,
            pltpu.SemaphoreType.DMA,
            pltpu.SemaphoreType.DMA,
        ],
    )


ROW_BLK = 1000


def _tc_body(x_ref, p0_ref, p1_ref, wroot_ref, wrel_ref, b_ref, s_ref,
             *out_refs):
    xb = x_ref[...]
    ab = p0_ref[...] + p1_ref[...]
    for level in range(NUM_LEVELS):
        xc = (jnp.dot(xb, wroot_ref[level], preferred_element_type=jnp.float32)
              + jnp.dot(ab, wrel_ref[level], preferred_element_type=jnp.float32)
              + b_ref[level][None, :])
        for j in range(3):
            p = level * 3 + j
            out_refs[p][...] = jnp.dot(xc, s_ref[p],
                                       preferred_element_type=jnp.float32)


def _tc_dense(x, p0, p1, Wroot, Wrel, b, S):
    grid = (N // ROW_BLK,)
    row_spec = pl.BlockSpec((ROW_BLK, D), lambda i: (i, 0))
    full = lambda shape: pl.BlockSpec(shape, lambda i: (0,) * len(shape))
    return pl.pallas_call(
        _tc_body,
        grid=grid,
        in_specs=[
            row_spec, row_spec, row_spec,
            full((NUM_LEVELS, D, H)),
            full((NUM_LEVELS, D, H)),
            full((NUM_LEVELS, H)),
            full((NUM_PROC, H, H)),
        ],
        out_specs=tuple(pl.BlockSpec((ROW_BLK, H), lambda i: (i, 0))
                        for _ in range(NUM_PROC)),
        out_shape=tuple(jax.ShapeDtypeStruct((N, H), jnp.float32)
                        for _ in range(NUM_PROC)),
    )(x, p0, p1, Wroot, Wrel, b, S)


def kernel(x, edge_index, Wroot, Wrel, b, S):
    pad = EPAD - E
    src = jnp.concatenate([edge_index[0], jnp.zeros((pad,), jnp.int32)])
    dst = jnp.concatenate([edge_index[1],
                           jnp.full((pad,), NP - 1, jnp.int32)])
    zz = jnp.zeros((NP, D), jnp.float32)
    parts = _sc_segment_sum()(x, src, dst, zz)
    outs = _tc_dense(x, parts[0, :N], parts[1, :N], Wroot, Wrel, b, S)
    return tuple(outs)


# speedup vs baseline: 1.0009x; 1.0009x over previous
import functools

import jax
import jax.numpy as jnp
from jax import lax
from jax.experimental import pallas as pl
from jax.experimental.pallas import tpu as pltpu
from jax.experimental.pallas import tpu_sc as plsc

N = 10000
E = 320000
D = 128
H = 128
NUM_LEVELS = 3
NUM_PROC = 9

NC = 2
NS = 16
NW = NC * NS
CH = 128
CPW = 80
EPAD = NW * CPW * CH
NP = 10240
ROWS_PER_TILE = NP // NS


def _sc_body(x_hbm, src_hbm, dst_hbm, zz_hbm, out_hbm,
             sidx0, sidx1, didx0, didx1, rows0, rows1, aggr_sh,
             semr0, semr1, semd0, semd1, sems0, sems1):
    c = lax.axis_index("c")
    s = lax.axis_index("s")
    wid = s * NC + c
    pltpu.sync_copy(zz_hbm.at[pl.ds(s * ROWS_PER_TILE, ROWS_PER_TILE)],
                    aggr_sh.at[pl.ds(s * ROWS_PER_TILE, ROWS_PER_TILE)])
    plsc.subcore_barrier()

    ebase = wid * CPW * CH
    rows = (rows0, rows1)
    sidx = (sidx0, sidx1)
    didx = (didx0, didx1)
    semr = (semr0, semr1)
    semd = (semd0, semd1)
    sems = (sems0, sems1)

    def idx_fetch(kk, b):
        pltpu.async_copy(src_hbm.at[pl.ds(ebase + kk * CH, CH)],
                         sidx[b], sems[b])
        pltpu.async_copy(dst_hbm.at[pl.ds(ebase + kk * CH, CH)],
                         didx[b], semd[b])

    idx_fetch(0, 0)
    idx_fetch(1, 1)
    pltpu.make_async_copy(src_hbm.at[pl.ds(ebase, CH)], sidx0, sems0).wait()
    pltpu.async_copy(x_hbm.at[sidx0], rows0, semr0)

    def step(i, carry):
        for b in range(2):
            kk = 2 * i + b
            nb = b ^ 1

            @pl.when(kk + 1 < CPW)
            def _():
                pltpu.make_async_copy(
                    src_hbm.at[pl.ds(ebase + (kk + 1) * CH, CH)],
                    sidx[nb], sems[nb]).wait()
                pltpu.async_copy(x_hbm.at[sidx[nb]], rows[nb], semr[nb])

            pltpu.make_async_copy(dst_hbm.at[pl.ds(ebase + kk * CH, CH)],
                                  didx[b], semd[b]).wait()
            pltpu.make_async_copy(x_hbm.at[sidx[b]], rows[b], semr[b]).wait()
            pltpu.sync_copy(rows[b], aggr_sh.at[didx[b]], add=True)

            @pl.when(kk + 2 < CPW)
            def _():
                idx_fetch(kk + 2, b)
        return carry

    lax.fori_loop(0, CPW // 2, step, 0)

    plsc.subcore_barrier()
    pltpu.sync_copy(aggr_sh.at[pl.ds(s * ROWS_PER_TILE, ROWS_PER_TILE)],
                    out_hbm.at[c, pl.ds(s * ROWS_PER_TILE, ROWS_PER_TILE)])


@functools.cache
def _sc_segment_sum():
    return pl.kernel(
        _sc_body,
        out_type=jax.ShapeDtypeStruct((NC, NP, D), jnp.float32),
        mesh=plsc.VectorSubcoreMesh(core_axis_name="c", subcore_axis_name="s",
                                    num_cores=NC, num_subcores=NS),
        scratch_types=[
            pltpu.VMEM((CH,), jnp.int32),
            pltpu.VMEM((CH,), jnp.int32),
            pltpu.VMEM((CH,), jnp.int32),
            pltpu.VMEM((CH,), jnp.int32),
            pltpu.VMEM((CH, D), jnp.float32),
            pltpu.VMEM((CH, D), jnp.float32),
            pltpu.VMEM_SHARED((NP, D), jnp.float32),
            pltpu.SemaphoreType.DMA,
            pltpu.SemaphoreType.DMA,
            pltpu.SemaphoreType.DMA,
            pltpu.SemaphoreType.DMA,
            pltpu.SemaphoreType.DMA,
            pltpu.SemaphoreType.DMA,
        ],
    )


ROW_BLK = 1000


def _tc_body(x_ref, p0_ref, p1_ref, wroot_ref, wrel_ref, b_ref, s_ref,
             *out_refs):
    xb = x_ref[...]
    ab = p0_ref[...] + p1_ref[...]
    for level in range(NUM_LEVELS):
        xc = (jnp.dot(xb, wroot_ref[level], preferred_element_type=jnp.float32)
              + jnp.dot(ab, wrel_ref[level], preferred_element_type=jnp.float32)
              + b_ref[level][None, :])
        for j in range(3):
            p = level * 3 + j
            out_refs[p][...] = jnp.dot(xc, s_ref[p],
                                       preferred_element_type=jnp.float32)


def _tc_dense(x, p0, p1, Wroot, Wrel, b, S):
    grid = (N // ROW_BLK,)
    row_spec = pl.BlockSpec((ROW_BLK, D), lambda i: (i, 0))
    full = lambda shape: pl.BlockSpec(shape, lambda i: (0,) * len(shape))
    return pl.pallas_call(
        _tc_body,
        grid=grid,
        in_specs=[
            row_spec, row_spec, row_spec,
            full((NUM_LEVELS, D, H)),
            full((NUM_LEVELS, D, H)),
            full((NUM_LEVELS, H)),
            full((NUM_PROC, H, H)),
        ],
        out_specs=tuple(pl.BlockSpec((ROW_BLK, H), lambda i: (i, 0))
                        for _ in range(NUM_PROC)),
        out_shape=tuple(jax.ShapeDtypeStruct((N, H), jnp.float32)
                        for _ in range(NUM_PROC)),
    )(x, p0, p1, Wroot, Wrel, b, S)


def kernel(x, edge_index, Wroot, Wrel, b, S):
    pad = EPAD - E
    src = jnp.concatenate([edge_index[0], jnp.zeros((pad,), jnp.int32)])
    pad_dst = N + (jnp.arange(pad, dtype=jnp.int32) % (NP - N))
    dst = jnp.concatenate([edge_index[1], pad_dst])
    zz = jnp.zeros((NP, D), jnp.float32)
    parts = _sc_segment_sum()(x, src, dst, zz)
    outs = _tc_dense(x, parts[0, :N], parts[1, :N], Wroot, Wrel, b, S)
    return tuple(outs)
